# baseline (speedup 1.0000x reference)
import functools

import jax
import jax.numpy as jnp
from jax import lax
from jax.experimental import pallas as pl
from jax.experimental.pallas import tpu as pltpu
from jax.experimental.pallas import tpu_sc as plsc

NUM_COLORS = 256
B, T, F = 64, 1024, 128
N = T * F
NW = 32
BPW = B // NW
TT = 64
CH = TT * F
NCH = T // TT
VPC = CH // 16
L = 16

_mesh = plsc.VectorSubcoreMesh(core_axis_name="c", subcore_axis_name="s")


def _minmax_body(x_ref, mn_ref, mx_ref):
    i = pl.program_id(0)
    blk = x_ref[...]
    mn_ref[i] = jnp.min(blk)
    mx_ref[i] = jnp.max(blk)


_minmax_tc = pl.pallas_call(
    _minmax_body,
    grid=(B,),
    in_specs=[pl.BlockSpec((1, T, F), lambda i: (i, 0, 0))],
    out_specs=[
        pl.BlockSpec(memory_space=pltpu.SMEM, block_shape=(B,),
                     index_map=lambda i: (0,)),
        pl.BlockSpec(memory_space=pltpu.SMEM, block_shape=(B,),
                     index_map=lambda i: (0,)),
    ],
    out_shape=[
        jax.ShapeDtypeStruct((B,), jnp.float32),
        jax.ShapeDtypeStruct((B,), jnp.float32),
    ],
)


@functools.partial(
    pl.kernel,
    mesh=_mesh,
    out_type=jax.ShapeDtypeStruct((B, T, 3, F), jnp.float32),
    scratch_types=[
        pltpu.VMEM((NUM_COLORS * 3,), jnp.float32),
        pltpu.VMEM((B,), jnp.float32),
        pltpu.VMEM((B,), jnp.float32),
        pltpu.VMEM((CH,), jnp.float32),
        pltpu.VMEM((CH,), jnp.float32),
        pltpu.VMEM((TT, 4, F), jnp.float32),
        pltpu.VMEM((TT, 4, F), jnp.float32),
        pltpu.SemaphoreType.DMA,
        pltpu.SemaphoreType.DMA,
        pltpu.SemaphoreType.DMA,
        pltpu.SemaphoreType.DMA,
    ],
    compiler_params=pltpu.CompilerParams(
        needs_layout_passes=False, disable_bounds_checks=True),
)
def _mel_to_rgb(x_hbm, lut_hbm, mn_hbm, mx_hbm, out_hbm,
                lut_v, mn_v, mx_v, xb0, xb1, ob0, ob1,
                si0, si1, so0, so1):
    cid = lax.axis_index("c")
    sid = lax.axis_index("s")
    wid = sid * 2 + cid

    xbs, obs = (xb0, xb1), (ob0, ob1)
    sis, sos = (si0, si1), (so0, so1)

    pltpu.sync_copy(lut_hbm, lut_v)
    pltpu.sync_copy(mn_hbm, mn_v)
    pltpu.sync_copy(mx_hbm, mx_v)

    for j in range(BPW):
        b = wid * BPW + j
        base = b * N

        bv = jnp.full((L,), b, jnp.int32)
        mnv = plsc.load_gather(mn_v, [bv])
        mxv = plsc.load_gather(mx_v, [bv])
        scale_v = (NUM_COLORS - 1) / (mxv - mnv + 1e-6)
        off_v = 0.5 - mnv * scale_v

        def start_in(ci, k):
            pltpu.async_copy(
                x_hbm.at[pl.ds(base + ci * CH, CH)], xbs[k], sis[k])

        def wait_in(k):
            pltpu.make_async_copy(
                x_hbm.at[pl.ds(base, CH)], xbs[k], sis[k]).wait()

        def start_out(ci, k):
            pltpu.async_copy(
                obs[k].at[:, pl.ds(0, 3), :],
                out_hbm.at[b, pl.ds(ci * TT, TT), pl.ds(0, 3)], sos[k])

        def wait_out(k):
            pltpu.make_async_copy(
                obs[k].at[:, pl.ds(0, 3), :],
                out_hbm.at[b, pl.ds(0, TT), pl.ds(0, 3)], sos[k]).wait()

        start_in(0, 0)
        start_in(1, 1)

        def pair_b(g, _):
            c0 = 2 * g
            for k in range(2):
                wait_in(k)

                @pl.when(c0 + k >= 2)
                def _():
                    wait_out(k)

                xbuf, obuf = xbs[k], obs[k]

                @plsc.parallel_loop(0, TT, 1, unroll=4)
                def row_b(trow):
                    for kk in range(F // L):
                        f0 = kk * L
                        xv = xbuf[pl.ds(trow * F + f0, L)]
                        t = xv * scale_v + off_v
                        idx = t.astype(jnp.int32)
                        idx = jnp.minimum(jnp.maximum(idx, 0), NUM_COLORS - 1)
                        r = plsc.load_gather(lut_v, [idx])
                        g2 = plsc.load_gather(lut_v, [idx + NUM_COLORS])
                        bl = plsc.load_gather(lut_v, [idx + 2 * NUM_COLORS])
                        obuf[trow, 0, pl.ds(f0, L)] = r
                        obuf[trow, 1, pl.ds(f0, L)] = g2
                        obuf[trow, 2, pl.ds(f0, L)] = bl

                start_out(c0 + k, k)

                @pl.when(c0 + 2 + k < NCH)
                def _():
                    start_in(c0 + 2 + k, k)
            return 0

        lax.fori_loop(0, NCH // 2, pair_b, 0)
        wait_out(0)
        wait_out(1)


def kernel(x, lut):
    lut_planar = lut.T.reshape(-1)
    mn, mx = _minmax_tc(x)
    out = _mel_to_rgb(x.reshape(-1), lut_planar, mn, mx)
    return out.transpose(0, 1, 3, 2)


# speedup vs baseline: 1.1782x; 1.1782x over previous
import functools

import jax
import jax.numpy as jnp
from jax import lax
from jax.experimental import pallas as pl
from jax.experimental.pallas import tpu as pltpu
from jax.experimental.pallas import tpu_sc as plsc

NUM_COLORS = 256
B, T, F = 64, 1024, 128
N = T * F
NW = 32
BPW = B // NW
TT = 64
CH = TT * F
NCH = T // TT
VPC = CH // 16
L = 16

_mesh = plsc.VectorSubcoreMesh(core_axis_name="c", subcore_axis_name="s")


_MMB = 8


def _minmax_body(x_ref, mn_ref, mx_ref):
    i = pl.program_id(0)
    for k in range(_MMB):
        blk = x_ref[k]
        mn_ref[i * _MMB + k] = jnp.min(blk)
        mx_ref[i * _MMB + k] = jnp.max(blk)


_minmax_tc = pl.pallas_call(
    _minmax_body,
    grid=(B // _MMB,),
    in_specs=[pl.BlockSpec((_MMB, T, F), lambda i: (i, 0, 0))],
    out_specs=[
        pl.BlockSpec(memory_space=pltpu.SMEM, block_shape=(B,),
                     index_map=lambda i: (0,)),
        pl.BlockSpec(memory_space=pltpu.SMEM, block_shape=(B,),
                     index_map=lambda i: (0,)),
    ],
    out_shape=[
        jax.ShapeDtypeStruct((B,), jnp.float32),
        jax.ShapeDtypeStruct((B,), jnp.float32),
    ],
)


@functools.partial(
    pl.kernel,
    mesh=_mesh,
    out_type=jax.ShapeDtypeStruct((B, T, 3, F), jnp.float32),
    scratch_types=[
        pltpu.VMEM((NUM_COLORS * 3,), jnp.float32),
        pltpu.VMEM((B,), jnp.float32),
        pltpu.VMEM((B,), jnp.float32),
        pltpu.VMEM((CH,), jnp.float32),
        pltpu.VMEM((CH,), jnp.float32),
        pltpu.VMEM((TT, 4, F), jnp.float32),
        pltpu.VMEM((TT, 4, F), jnp.float32),
        pltpu.SemaphoreType.DMA,
        pltpu.SemaphoreType.DMA,
        pltpu.SemaphoreType.DMA,
        pltpu.SemaphoreType.DMA,
    ],
    compiler_params=pltpu.CompilerParams(
        needs_layout_passes=False, disable_bounds_checks=True),
)
def _mel_to_rgb(x_hbm, lut_hbm, mn_hbm, mx_hbm, out_hbm,
                lut_v, mn_v, mx_v, xb0, xb1, ob0, ob1,
                si0, si1, so0, so1):
    cid = lax.axis_index("c")
    sid = lax.axis_index("s")
    wid = sid * 2 + cid

    xbs, obs = (xb0, xb1), (ob0, ob1)
    sis, sos = (si0, si1), (so0, so1)

    pltpu.sync_copy(lut_hbm, lut_v)
    pltpu.sync_copy(mn_hbm, mn_v)
    pltpu.sync_copy(mx_hbm, mx_v)

    for j in range(BPW):
        b = wid * BPW + j
        base = b * N

        bv = jnp.full((L,), b, jnp.int32)
        mnv = plsc.load_gather(mn_v, [bv])
        mxv = plsc.load_gather(mx_v, [bv])
        scale_v = (NUM_COLORS - 1) / (mxv - mnv + 1e-6)
        off_v = 0.5 - mnv * scale_v

        def start_in(ci, k):
            pltpu.async_copy(
                x_hbm.at[pl.ds(base + ci * CH, CH)], xbs[k], sis[k])

        def wait_in(k):
            pltpu.make_async_copy(
                x_hbm.at[pl.ds(base, CH)], xbs[k], sis[k]).wait()

        def start_out(ci, k):
            pltpu.async_copy(
                obs[k].at[:, pl.ds(0, 3), :],
                out_hbm.at[b, pl.ds(ci * TT, TT), pl.ds(0, 3)], sos[k])

        def wait_out(k):
            pltpu.make_async_copy(
                obs[k].at[:, pl.ds(0, 3), :],
                out_hbm.at[b, pl.ds(0, TT), pl.ds(0, 3)], sos[k]).wait()

        start_in(0, 0)
        start_in(1, 1)

        def pair_b(g, _):
            c0 = 2 * g
            for k in range(2):
                wait_in(k)

                @pl.when(c0 + k >= 2)
                def _():
                    wait_out(k)

                xbuf, obuf = xbs[k], obs[k]

                @plsc.parallel_loop(0, TT, 1, unroll=4)
                def row_b(trow):
                    for kk in range(F // L):
                        f0 = kk * L
                        xv = xbuf[pl.ds(trow * F + f0, L)]
                        t = xv * scale_v + off_v
                        idx = t.astype(jnp.int32)
                        idx = jnp.minimum(jnp.maximum(idx, 0), NUM_COLORS - 1)
                        r = plsc.load_gather(lut_v, [idx])
                        g2 = plsc.load_gather(lut_v, [idx + NUM_COLORS])
                        bl = plsc.load_gather(lut_v, [idx + 2 * NUM_COLORS])
                        obuf[trow, 0, pl.ds(f0, L)] = r
                        obuf[trow, 1, pl.ds(f0, L)] = g2
                        obuf[trow, 2, pl.ds(f0, L)] = bl

                start_out(c0 + k, k)

                @pl.when(c0 + 2 + k < NCH)
                def _():
                    start_in(c0 + 2 + k, k)
            return 0

        lax.fori_loop(0, NCH // 2, pair_b, 0)
        wait_out(0)
        wait_out(1)


def kernel(x, lut):
    lut_planar = lut.T.reshape(-1)
    mn, mx = _minmax_tc(x)
    out = _mel_to_rgb(x.reshape(-1), lut_planar, mn, mx)
    return out.transpose(0, 1, 3, 2)


# speedup vs baseline: 1.1848x; 1.0056x over previous
import functools

import jax
import jax.numpy as jnp
from jax import lax
from jax.experimental import pallas as pl
from jax.experimental.pallas import tpu as pltpu
from jax.experimental.pallas import tpu_sc as plsc

NUM_COLORS = 256
B, T, F = 64, 1024, 128
N = T * F
NW = 32
BPW = B // NW
TT = 64
CH = TT * F
NCH = T // TT
VPC = CH // 16
L = 16

_mesh = plsc.VectorSubcoreMesh(core_axis_name="c", subcore_axis_name="s")


_MMB = 16


def _minmax_body(x_ref, mn_ref, mx_ref):
    i = pl.program_id(0)
    for k in range(_MMB):
        blk = x_ref[k]
        mn_ref[i * _MMB + k] = jnp.min(blk)
        mx_ref[i * _MMB + k] = jnp.max(blk)


_minmax_tc = pl.pallas_call(
    _minmax_body,
    grid=(B // _MMB,),
    in_specs=[pl.BlockSpec((_MMB, T, F), lambda i: (i, 0, 0))],
    out_specs=[
        pl.BlockSpec(memory_space=pltpu.SMEM, block_shape=(B,),
                     index_map=lambda i: (0,)),
        pl.BlockSpec(memory_space=pltpu.SMEM, block_shape=(B,),
                     index_map=lambda i: (0,)),
    ],
    out_shape=[
        jax.ShapeDtypeStruct((B,), jnp.float32),
        jax.ShapeDtypeStruct((B,), jnp.float32),
    ],
)


@functools.partial(
    pl.kernel,
    mesh=_mesh,
    out_type=jax.ShapeDtypeStruct((B, T, 3, F), jnp.float32),
    scratch_types=[
        pltpu.VMEM((NUM_COLORS * 3,), jnp.float32),
        pltpu.VMEM((B,), jnp.float32),
        pltpu.VMEM((B,), jnp.float32),
        pltpu.VMEM((CH,), jnp.float32),
        pltpu.VMEM((CH,), jnp.float32),
        pltpu.VMEM((TT, 4, F), jnp.float32),
        pltpu.VMEM((TT, 4, F), jnp.float32),
        pltpu.SemaphoreType.DMA,
        pltpu.SemaphoreType.DMA,
        pltpu.SemaphoreType.DMA,
        pltpu.SemaphoreType.DMA,
    ],
    compiler_params=pltpu.CompilerParams(
        needs_layout_passes=False, disable_bounds_checks=True),
)
def _mel_to_rgb(x_hbm, lut_hbm, mn_hbm, mx_hbm, out_hbm,
                lut_v, mn_v, mx_v, xb0, xb1, ob0, ob1,
                si0, si1, so0, so1):
    cid = lax.axis_index("c")
    sid = lax.axis_index("s")
    wid = sid * 2 + cid

    xbs, obs = (xb0, xb1), (ob0, ob1)
    sis, sos = (si0, si1), (so0, so1)

    pltpu.sync_copy(lut_hbm, lut_v)
    pltpu.sync_copy(mn_hbm, mn_v)
    pltpu.sync_copy(mx_hbm, mx_v)

    for j in range(BPW):
        b = wid * BPW + j
        base = b * N

        bv = jnp.full((L,), b, jnp.int32)
        mnv = plsc.load_gather(mn_v, [bv])
        mxv = plsc.load_gather(mx_v, [bv])
        scale_v = (NUM_COLORS - 1) / (mxv - mnv + 1e-6)
        off_v = 0.5 - mnv * scale_v

        def start_in(ci, k):
            pltpu.async_copy(
                x_hbm.at[pl.ds(base + ci * CH, CH)], xbs[k], sis[k])

        def wait_in(k):
            pltpu.make_async_copy(
                x_hbm.at[pl.ds(base, CH)], xbs[k], sis[k]).wait()

        def start_out(ci, k):
            pltpu.async_copy(
                obs[k].at[:, pl.ds(0, 3), :],
                out_hbm.at[b, pl.ds(ci * TT, TT), pl.ds(0, 3)], sos[k])

        def wait_out(k):
            pltpu.make_async_copy(
                obs[k].at[:, pl.ds(0, 3), :],
                out_hbm.at[b, pl.ds(0, TT), pl.ds(0, 3)], sos[k]).wait()

        start_in(0, 0)
        start_in(1, 1)

        def pair_b(g, _):
            c0 = 2 * g
            for k in range(2):
                wait_in(k)

                @pl.when(c0 + k >= 2)
                def _():
                    wait_out(k)

                xbuf, obuf = xbs[k], obs[k]

                @plsc.parallel_loop(0, TT, 1, unroll=4)
                def row_b(trow):
                    for kk in range(F // L):
                        f0 = kk * L
                        xv = xbuf[pl.ds(trow * F + f0, L)]
                        t = xv * scale_v + off_v
                        idx = t.astype(jnp.int32)
                        idx = jnp.minimum(jnp.maximum(idx, 0), NUM_COLORS - 1)
                        r = plsc.load_gather(lut_v, [idx])
                        g2 = plsc.load_gather(lut_v, [idx + NUM_COLORS])
                        bl = plsc.load_gather(lut_v, [idx + 2 * NUM_COLORS])
                        obuf[trow, 0, pl.ds(f0, L)] = r
                        obuf[trow, 1, pl.ds(f0, L)] = g2
                        obuf[trow, 2, pl.ds(f0, L)] = bl

                start_out(c0 + k, k)

                @pl.when(c0 + 2 + k < NCH)
                def _():
                    start_in(c0 + 2 + k, k)
            return 0

        lax.fori_loop(0, NCH // 2, pair_b, 0)
        wait_out(0)
        wait_out(1)


def kernel(x, lut):
    lut_planar = lut.T.reshape(-1)
    mn, mx = _minmax_tc(x)
    out = _mel_to_rgb(x.reshape(-1), lut_planar, mn, mx)
    return out.transpose(0, 1, 3, 2)
